# Initial kernel scaffold; baseline (speedup 1.0000x reference)
import jax
import jax.numpy as jnp
from jax.experimental import pallas as pl


def kernel(data, table):
    raise NotImplementedError("write your pallas kernel here")


# speedup vs baseline: 1.5470x; 1.5470x over previous
import functools

import jax
import jax.numpy as jnp
from jax import lax
from jax.experimental import pallas as pl
from jax.experimental.pallas import tpu as pltpu
from jax.experimental.pallas import tpu_sc as plsc


def _make_gather(V, D, B):
    info = plsc.get_sparse_core_info()
    NC, NS = info.num_cores, info.num_subcores
    NW = NC * NS
    assert B % NW == 0
    b_per_w = B // NW
    CHUNK = 1024
    assert b_per_w % CHUNK == 0
    n_chunks = b_per_w // CHUNK
    SUB = 128
    n_sub = CHUNK // SUB
    mesh = plsc.VectorSubcoreMesh(core_axis_name="c", subcore_axis_name="s")

    @functools.partial(
        pl.kernel,
        mesh=mesh,
        out_type=jax.ShapeDtypeStruct((B, D), jnp.float32),
        scratch_types=[
            pltpu.VMEM((CHUNK,), jnp.int32),
            pltpu.VMEM((CHUNK, D), jnp.float32),
            pltpu.SemaphoreType.DMA,
        ],
        compiler_params=pltpu.CompilerParams(use_tc_tiling_on_sc=False),
    )
    def gather_kernel(table_hbm, idx_hbm, out_hbm, idx_v, rows_v, sem):
        wid = lax.axis_index("s") * NC + lax.axis_index("c")
        base = wid * b_per_w

        def body(i, carry):
            off = base + i * CHUNK
            pltpu.sync_copy(idx_hbm.at[pl.ds(off, CHUNK)], idx_v)
            copies = []
            for j in range(n_sub):
                copies.append(
                    pltpu.async_copy(
                        table_hbm.at[idx_v.at[pl.ds(j * SUB, SUB)]],
                        rows_v.at[pl.ds(j * SUB, SUB)],
                        sem,
                    )
                )
            for c in copies:
                c.wait()
            pltpu.sync_copy(rows_v, out_hbm.at[pl.ds(off, CHUNK)])
            return carry

        lax.fori_loop(0, n_chunks, body, 0)

    return gather_kernel


def kernel(data, table):
    B, F = data.shape
    V, D = table.shape
    idx = data.reshape(-1).astype(jnp.int32)
    out = _make_gather(V, D, B * F)(table, idx)
    return out.reshape(B, F, D)


# speedup vs baseline: 1.5753x; 1.0184x over previous
import functools

import jax
import jax.numpy as jnp
from jax import lax
from jax.experimental import pallas as pl
from jax.experimental.pallas import tpu as pltpu
from jax.experimental.pallas import tpu_sc as plsc

SUB = 128
CHUNK = 256
NBUF = 4
AHEAD = 3


def _make_gather(V, D, B):
    info = plsc.get_sparse_core_info()
    NC, NS = info.num_cores, info.num_subcores
    NW = NC * NS
    assert B % NW == 0
    b_per_w = B // NW
    n_sub = CHUNK // SUB
    assert b_per_w % (CHUNK * NBUF) == 0
    n_chunks = b_per_w // CHUNK
    n_groups = n_chunks // NBUF
    mesh = plsc.VectorSubcoreMesh(core_axis_name="c", subcore_axis_name="s")

    @functools.partial(
        pl.kernel,
        mesh=mesh,
        out_type=jax.ShapeDtypeStruct((B, D), jnp.float32),
        scratch_types=[
            pltpu.VMEM((b_per_w,), jnp.int32),
            pltpu.VMEM((NBUF, CHUNK, D), jnp.float32),
            [pltpu.SemaphoreType.DMA] * NBUF,
            [pltpu.SemaphoreType.DMA] * NBUF,
        ],
        compiler_params=pltpu.CompilerParams(use_tc_tiling_on_sc=False),
    )
    def gather_kernel(table_hbm, idx_hbm, out_hbm, idx_all, rows_v, sem_g, sem_o):
        wid = lax.axis_index("s") * NC + lax.axis_index("c")
        base = wid * b_per_w
        pltpu.sync_copy(idx_hbm.at[pl.ds(base, b_per_w)], idx_all)

        def fire(c, b):
            for j in range(n_sub):
                pltpu.async_copy(
                    table_hbm.at[idx_all.at[pl.ds(c * CHUNK + j * SUB, SUB)]],
                    rows_v.at[b].at[pl.ds(j * SUB, SUB)],
                    sem_g[b],
                )

        def drain_gathers(b):
            pltpu.make_async_copy(
                out_hbm.at[pl.ds(base, CHUNK)], rows_v.at[b], sem_g[b]
            ).wait()

        def start_out(c, b):
            pltpu.async_copy(
                rows_v.at[b], out_hbm.at[pl.ds(base + c * CHUNK, CHUNK)], sem_o[b]
            )

        def wait_out(b):
            pltpu.make_async_copy(
                rows_v.at[b], out_hbm.at[pl.ds(base, CHUNK)], sem_o[b]
            ).wait()

        for c0 in range(AHEAD):
            fire(c0, c0 % NBUF)

        def body(g, carry):
            for b in range(NBUF):
                c = g * NBUF + b
                @pl.when(jnp.logical_and(c + AHEAD < n_chunks, c + AHEAD >= NBUF))
                def _():
                    wait_out((b + AHEAD) % NBUF)

                @pl.when(c + AHEAD < n_chunks)
                def _():
                    fire(c + AHEAD, (b + AHEAD) % NBUF)

                drain_gathers(b)
                start_out(c, b)
            return carry

        lax.fori_loop(0, n_groups, body, 0)
        for b in range(NBUF):
            wait_out(b)

    return gather_kernel


def kernel(data, table):
    B, F = data.shape
    V, D = table.shape
    idx = data.reshape(-1).astype(jnp.int32)
    out = _make_gather(V, D, B * F)(table, idx)
    return out.reshape(B, F, D)
